# baseline (speedup 1.0000x reference)
import functools

import jax
import jax.numpy as jnp
from jax import lax
from jax.experimental import pallas as pl
from jax.experimental.pallas import tpu as pltpu
from jax.experimental.pallas import tpu_sc as plsc

M = 1000000
D = 64
B = 16384
W = M + 16 * B

_info = plsc.get_sparse_core_info()
NC, NS, L = _info.num_cores, _info.num_subcores, _info.num_lanes
NW = NC * NS
CHUNK = B // NW

_mesh = plsc.VectorSubcoreMesh(core_axis_name="c", subcore_axis_name="s")


def _wid_base():
    wid = lax.axis_index("s") * NC + lax.axis_index("c")
    return wid, wid * CHUNK


ROUNDS = 6


@functools.partial(
    pl.kernel,
    mesh=_mesh,
    out_type=jax.ShapeDtypeStruct((NW, L), jnp.int32),
    scratch_types=[
        pltpu.VMEM((CHUNK,), jnp.int32),
        pltpu.VMEM((CHUNK,), jnp.int32),
        pltpu.VMEM((CHUNK,), jnp.int32),
        pltpu.VMEM((CHUNK,), jnp.int32),
        pltpu.VMEM((L,), jnp.int32),
        pltpu.SemaphoreType.DMA,
    ],
)
def _resolve(idx_hbm, win_hbm, flags_hbm, idx_v, w_v, j_v, tgt_v, flag_v, sem):
    wid, base = _wid_base()
    pltpu.sync_copy(idx_hbm.at[pl.ds(base, CHUNK)], idx_v)
    for k in range(CHUNK // L):
        j_v[pl.ds(k * L, L)] = lax.iota(jnp.int32, L) + (base + k * L)

    for _ in range(ROUNDS):
        pltpu.async_copy(win_hbm.at[idx_v], w_v, sem).wait()
        acc = jnp.zeros((L,), jnp.int32)
        for k in range(CHUNK // L):
            j = j_v[pl.ds(k * L, L)]
            w = w_v[pl.ds(k * L, L)]
            viol = j > w
            acc = acc + jnp.where(viol, jnp.int32(1), jnp.int32(0))
            tgt_v[pl.ds(k * L, L)] = jnp.where(
                viol, idx_v[pl.ds(k * L, L)], j * jnp.int32(16) + jnp.int32(M)
            )
        pltpu.async_copy(j_v, win_hbm.at[tgt_v], sem).wait()
        plsc.subcore_barrier()
    flag_v[...] = acc
    pltpu.sync_copy(flag_v, flags_hbm.at[wid])


@functools.partial(
    pl.kernel,
    mesh=_mesh,
    out_type=jax.ShapeDtypeStruct((B, D), jnp.float32),
    scratch_types=[
        pltpu.VMEM((CHUNK,), jnp.int32),
        pltpu.VMEM((CHUNK,), jnp.int32),
        pltpu.VMEM((CHUNK, D), jnp.float32),
        pltpu.SemaphoreType.DMA,
    ],
    compiler_params=pltpu.CompilerParams(use_tc_tiling_on_sc=False),
)
def _gather_rows(idx_hbm, win_hbm, val_hbm, out_hbm, idx_v, w_v, rows_v, sem):
    _, base = _wid_base()
    pltpu.sync_copy(idx_hbm.at[pl.ds(base, CHUNK)], idx_v)
    pltpu.async_copy(win_hbm.at[idx_v], w_v, sem).wait()
    pltpu.async_copy(val_hbm.at[w_v], rows_v, sem).wait()
    pltpu.sync_copy(rows_v, out_hbm.at[pl.ds(base, CHUNK)])


def kernel(mem, idx, val):
    del mem
    idx32 = idx.astype(jnp.int32)
    win_ref = jax.new_ref(jnp.zeros((W,), jnp.int32))
    flags = _resolve(idx32, win_ref)

    def cond(f):
        return f > 0

    def body(_):
        return jnp.sum(_resolve(idx32, win_ref))

    lax.while_loop(cond, body, jnp.sum(flags))
    return _gather_rows(idx32, win_ref, val)


# speedup vs baseline: 1.9240x; 1.9240x over previous
import functools

import jax
import jax.numpy as jnp
from jax import lax
from jax.experimental import pallas as pl
from jax.experimental.pallas import tpu as pltpu
from jax.experimental.pallas import tpu_sc as plsc

M = 1000000
D = 64
B = 16384
W = M + 16 * B

_info = plsc.get_sparse_core_info()
NC, NS, L = _info.num_cores, _info.num_subcores, _info.num_lanes
NW = NC * NS
CHUNK = B // NW

_mesh = plsc.VectorSubcoreMesh(core_axis_name="c", subcore_axis_name="s")


def _wid_base():
    wid = lax.axis_index("s") * NC + lax.axis_index("c")
    return wid, wid * CHUNK


@functools.partial(
    pl.kernel,
    mesh=_mesh,
    out_type=jax.ShapeDtypeStruct((NW, L), jnp.int32),
    scratch_types=[
        pltpu.VMEM((CHUNK,), jnp.int32),
        pltpu.VMEM((CHUNK,), jnp.int32),
        pltpu.VMEM((CHUNK,), jnp.int32),
        pltpu.VMEM((CHUNK,), jnp.int32),
        pltpu.VMEM((L,), jnp.int32),
        pltpu.SemaphoreType.DMA,
    ],
)
def _fix_round(idx_hbm, win_hbm, flags_hbm, idx_v, w_v, new_v, tgt_v, flag_v, sem):
    wid, base = _wid_base()
    pltpu.sync_copy(idx_hbm.at[pl.ds(base, CHUNK)], idx_v)
    pltpu.async_copy(win_hbm.at[idx_v], w_v, sem).wait()
    acc = jnp.zeros((L,), jnp.int32)
    for k in range(CHUNK // L):
        j = lax.iota(jnp.int32, L) + (base + k * L)
        dummy = j * 16 + M
        w = w_v[pl.ds(k * L, L)]
        viol = j > w
        acc = acc + jnp.where(viol, jnp.int32(1), jnp.int32(0))
        new_v[pl.ds(k * L, L)] = j
        tgt_v[pl.ds(k * L, L)] = jnp.where(viol, idx_v[pl.ds(k * L, L)], dummy)
    flag_v[...] = acc
    pltpu.async_copy(new_v, win_hbm.at[tgt_v], sem).wait()
    pltpu.sync_copy(flag_v, flags_hbm.at[wid])


@functools.partial(
    pl.kernel,
    mesh=_mesh,
    out_type=jax.ShapeDtypeStruct((B, D), jnp.float32),
    scratch_types=[
        pltpu.VMEM((CHUNK,), jnp.int32),
        pltpu.VMEM((CHUNK,), jnp.int32),
        pltpu.VMEM((CHUNK, D), jnp.float32),
        pltpu.SemaphoreType.DMA,
    ],
    compiler_params=pltpu.CompilerParams(use_tc_tiling_on_sc=False),
)
def _gather_rows(idx_hbm, win_hbm, val_hbm, out_hbm, idx_v, w_v, rows_v, sem):
    _, base = _wid_base()
    pltpu.sync_copy(idx_hbm.at[pl.ds(base, CHUNK)], idx_v)
    pltpu.async_copy(win_hbm.at[idx_v], w_v, sem).wait()
    pltpu.async_copy(val_hbm.at[w_v], rows_v, sem).wait()
    pltpu.sync_copy(rows_v, out_hbm.at[pl.ds(base, CHUNK)])


def kernel(mem, idx, val):
    del mem
    idx32 = idx.astype(jnp.int32)
    win_ref = jax.new_ref(jnp.zeros((W,), jnp.int32))
    for _ in range(4):
        flags = _fix_round(idx32, win_ref)

    def cond(f):
        return f > 0

    def body(_):
        return jnp.sum(_fix_round(idx32, win_ref))

    lax.while_loop(cond, body, jnp.sum(flags))
    return _gather_rows(idx32, win_ref, val)
